# baseline (device time: 456923 ns/iter reference)
import jax
import jax.numpy as jnp
from jax import lax
from jax.experimental import pallas as pl
from jax.experimental.pallas import tpu as pltpu

N_DEV = 16
T_BLK = 16
T_CORR = 48


def kernel(x, A, B, C):
    b, s, d = x.shape
    n = A.shape[1]
    nblk = s // T_BLK

    cdt = jnp.bfloat16
    xt = jnp.transpose(x, (1, 0, 2)).astype(cdt)
    Bt = jnp.transpose(B, (1, 0, 2)).astype(cdt)
    Ct = jnp.transpose(C, (1, 0, 2)).astype(cdt)
    dA = jnp.exp(A.T)
    dAinv = jnp.exp(-A.T)

    def body(x_ref, dA_ref, dAinv_ref, B_ref, C_ref, out_ref,
             h_ref, recv_ref, pow_ref, inv16_ref, send_sem, recv_sem):
        my_i = lax.axis_index("i")
        left = (my_i - 1) % N_DEV
        right = (my_i + 1) % N_DEV

        barrier_sem = pltpu.get_barrier_semaphore()
        for nbr in (left, right):
            pl.semaphore_signal(
                barrier_sem, inc=1,
                device_id=(nbr,), device_id_type=pl.DeviceIdType.MESH,
            )
        pl.semaphore_wait(barrier_sem, 2)

        pow_ref[0] = jnp.ones((n, d), jnp.float32)
        inv16_ref[0] = jnp.ones((n, d), cdt)
        inv_f32 = jnp.ones((n, d), jnp.float32)
        for j in range(1, T_BLK):
            pow_ref[j] = pow_ref[j - 1] * dA_ref[...]
            inv_f32 = inv_f32 * dAinv_ref[...]
            inv16_ref[j] = inv_f32.astype(cdt)

        row = lax.broadcasted_iota(jnp.int32, (T_BLK, T_BLK), 0)
        col = lax.broadcasted_iota(jnp.int32, (T_BLK, T_BLK), 1)
        L = (row >= col).astype(cdt)

        h_ref[...] = jnp.zeros_like(h_ref)

        def block(k, carry):
            t0 = k * T_BLK
            xb = x_ref[pl.ds(t0, T_BLK)]
            Bb = B_ref[pl.ds(t0, T_BLK)]
            Cb = C_ref[pl.ds(t0, T_BLK)]

            Q = (xb[:, :, None, :] * Bb[:, :, :, None]) \
                * inv16_ref[...][:, None]
            S = jnp.tensordot(
                L, Q, axes=((1,), (0,)),
                preferred_element_type=jnp.float32)
            dAh0 = dA_ref[...][None] * h_ref[...]
            W = (S + dAh0[None]) * pow_ref[...][:, None]
            y = jnp.sum(W * Cb[:, :, :, None].astype(jnp.float32),
                        axis=2)
            out_ref[pl.ds(t0, T_BLK)] = y.astype(cdt)
            h_ref[...] = W[T_BLK - 1]
            return carry

        lax.fori_loop(0, nblk, block, 0)

        rdma = pltpu.make_async_remote_copy(
            src_ref=h_ref,
            dst_ref=recv_ref,
            send_sem=send_sem,
            recv_sem=recv_sem,
            device_id=(right,),
            device_id_type=pl.DeviceIdType.MESH,
        )
        rdma.start()
        rdma.wait()

        @pl.when(my_i != 0)
        def _():
            def corr(t, carry):
                g = recv_ref[...] * dA_ref[...][None]
                recv_ref[...] = g
                out_ref[t] = out_ref[t] + jnp.sum(
                    g * C_ref[t][:, :, None], axis=1).astype(cdt)
                return carry

            lax.fori_loop(0, T_CORR, corr, 0)

    out = pl.pallas_call(
        body,
        out_shape=jax.ShapeDtypeStruct((s, b, d), cdt),
        in_specs=[pl.BlockSpec(memory_space=pltpu.VMEM)] * 5,
        out_specs=pl.BlockSpec(memory_space=pltpu.VMEM),
        scratch_shapes=[
            pltpu.VMEM((b, n, d), jnp.float32),
            pltpu.VMEM((b, n, d), jnp.float32),
            pltpu.VMEM((T_BLK, n, d), jnp.float32),
            pltpu.VMEM((T_BLK, n, d), cdt),
            pltpu.SemaphoreType.DMA,
            pltpu.SemaphoreType.DMA,
        ],
        compiler_params=pltpu.CompilerParams(collective_id=0),
    )(xt, dA, dAinv, Bt, Ct)

    return jnp.transpose(out, (1, 0, 2)).astype(jnp.float32)


# device time: 343968 ns/iter; 1.3284x vs baseline; 1.3284x over previous
import jax
import jax.numpy as jnp
from jax import lax
from jax.experimental import pallas as pl
from jax.experimental.pallas import tpu as pltpu

N_DEV = 16
T_CORR = 48


def kernel(x, A, B, C):
    b, s, d = x.shape
    n = A.shape[1]
    r = b * n

    cdt = jnp.bfloat16
    eye = jnp.eye(b, dtype=jnp.float32)

    xt = jnp.transpose(x, (1, 0, 2)).astype(cdt)
    Bt = jnp.transpose(B, (1, 0, 2))
    Ct = jnp.transpose(C, (1, 0, 2))
    Bcol = jnp.einsum("sbn,bc->sbnc", Bt, eye).reshape(s, r, b).astype(cdt)
    M = jnp.einsum("sbn,pb->spbn", Ct, eye).reshape(s, b, r).astype(cdt)
    dA2 = jnp.tile(jnp.exp(A.T), (b, 1)).astype(cdt)

    def body(x_ref, dA2_ref, Bcol_ref, M_ref, out_ref,
             h_ref, recv_ref, send_sem, recv_sem):
        my_i = lax.axis_index("i")
        left = (my_i - 1) % N_DEV
        right = (my_i + 1) % N_DEV

        barrier_sem = pltpu.get_barrier_semaphore()
        for nbr in (left, right):
            pl.semaphore_signal(
                barrier_sem, inc=1,
                device_id=(nbr,), device_id_type=pl.DeviceIdType.MESH,
            )
        pl.semaphore_wait(barrier_sem, 2)

        dAv = dA2_ref[...]
        h_ref[...] = jnp.zeros_like(h_ref)

        def step(t, carry):
            u = jnp.dot(Bcol_ref[t], x_ref[t],
                        preferred_element_type=jnp.float32)
            h = h_ref[...] * dAv + u.astype(cdt)
            h_ref[...] = h
            y = jnp.dot(M_ref[t], h,
                        preferred_element_type=jnp.float32)
            out_ref[t] = y.astype(cdt)
            return carry

        lax.fori_loop(0, s, step, 0, unroll=8)

        rdma = pltpu.make_async_remote_copy(
            src_ref=h_ref,
            dst_ref=recv_ref,
            send_sem=send_sem,
            recv_sem=recv_sem,
            device_id=(right,),
            device_id_type=pl.DeviceIdType.MESH,
        )
        rdma.start()
        rdma.wait()

        @pl.when(my_i != 0)
        def _():
            def corr(t, carry):
                g = recv_ref[...] * dAv
                recv_ref[...] = g
                yc = jnp.dot(M_ref[t], g,
                             preferred_element_type=jnp.float32)
                out_ref[t] = out_ref[t] + yc.astype(cdt)
                return carry

            lax.fori_loop(0, T_CORR, corr, 0)

    out = pl.pallas_call(
        body,
        out_shape=jax.ShapeDtypeStruct((s, b, d), cdt),
        in_specs=[pl.BlockSpec(memory_space=pltpu.VMEM)] * 4,
        out_specs=pl.BlockSpec(memory_space=pltpu.VMEM),
        scratch_shapes=[
            pltpu.VMEM((r, d), cdt),
            pltpu.VMEM((r, d), cdt),
            pltpu.SemaphoreType.DMA,
            pltpu.SemaphoreType.DMA,
        ],
        compiler_params=pltpu.CompilerParams(
            collective_id=0, vmem_limit_bytes=56 * 1024 * 1024),
    )(xt, dA2, Bcol, M)

    return jnp.transpose(out, (1, 0, 2)).astype(jnp.float32)


# device time: 240986 ns/iter; 1.8961x vs baseline; 1.4273x over previous
import jax
import jax.numpy as jnp
from jax import lax
from jax.experimental import pallas as pl
from jax.experimental.pallas import tpu as pltpu

N_DEV = 16
T_BLK = 8
T_CORR = 48


def kernel(x, A, B, C):
    b, s, d = x.shape
    n = A.shape[1]

    cdt = jnp.bfloat16
    xt = jnp.transpose(x, (1, 0, 2)).astype(cdt)
    Bt = jnp.transpose(B, (1, 0, 2)).astype(cdt)
    Ct = jnp.transpose(C, (1, 0, 2)).astype(cdt)
    dA = jnp.exp(A.T).astype(cdt)

    def body(x_ref, dA_ref, B_ref, C_ref, out_ref,
             h_ref, recv_ref, send_sem, recv_sem):
        my_i = lax.axis_index("i")
        left = (my_i - 1) % N_DEV
        right = (my_i + 1) % N_DEV

        barrier_sem = pltpu.get_barrier_semaphore()
        for nbr in (left, right):
            pl.semaphore_signal(
                barrier_sem, inc=1,
                device_id=(nbr,), device_id_type=pl.DeviceIdType.MESH,
            )
        pl.semaphore_wait(barrier_sem, 2)

        dAv = dA_ref[...][None]
        h_ref[...] = jnp.zeros_like(h_ref)

        def blk(k, carry):
            t0 = k * T_BLK
            xb = x_ref[pl.ds(t0, T_BLK)]
            Bb = B_ref[pl.ds(t0, T_BLK)]
            Cb = C_ref[pl.ds(t0, T_BLK)]
            h = h_ref[...]
            ys = []
            for j in range(T_BLK):
                h = h * dAv + xb[j][:, None, :] * Bb[j][:, :, None]
                ys.append(jnp.sum(h * Cb[j][:, :, None], axis=1,
                                  dtype=jnp.float32).astype(cdt))
            h_ref[...] = h
            out_ref[pl.ds(t0, T_BLK)] = jnp.stack(ys, axis=0)
            return carry

        lax.fori_loop(0, s // T_BLK, blk, 0)

        rdma = pltpu.make_async_remote_copy(
            src_ref=h_ref,
            dst_ref=recv_ref,
            send_sem=send_sem,
            recv_sem=recv_sem,
            device_id=(right,),
            device_id_type=pl.DeviceIdType.MESH,
        )
        rdma.start()
        rdma.wait()

        @pl.when(my_i != 0)
        def _():
            def corr(t, carry):
                g = recv_ref[...] * dAv
                recv_ref[...] = g
                out_ref[t] = out_ref[t] + jnp.sum(
                    g * C_ref[t][:, :, None], axis=1,
                    dtype=jnp.float32).astype(cdt)
                return carry

            lax.fori_loop(0, T_CORR, corr, 0)

    out = pl.pallas_call(
        body,
        out_shape=jax.ShapeDtypeStruct((s, b, d), cdt),
        in_specs=[pl.BlockSpec(memory_space=pltpu.VMEM)] * 4,
        out_specs=pl.BlockSpec(memory_space=pltpu.VMEM),
        scratch_shapes=[
            pltpu.VMEM((b, n, d), cdt),
            pltpu.VMEM((b, n, d), cdt),
            pltpu.SemaphoreType.DMA,
            pltpu.SemaphoreType.DMA,
        ],
        compiler_params=pltpu.CompilerParams(
            collective_id=0, vmem_limit_bytes=56 * 1024 * 1024),
    )(xt, dA, Bt, Ct)

    return jnp.transpose(out, (1, 0, 2)).astype(jnp.float32)
